# baseline (speedup 1.0000x reference)
import functools

import jax
import jax.numpy as jnp
from jax import lax
from jax.experimental import pallas as pl
from jax.experimental.pallas import tpu as pltpu
from jax.experimental.pallas import tpu_sc as plsc

B = 16384
NF = 32
EPS = 1e-5

_NC = 2
_NS = 16
_NW = _NC * _NS
_BPW = B // _NW
_CHUNK = 128
_NCHUNK = _BPW // _CHUNK


def _sc_gather4(ue_gmf, ee_gmf, ue_mlp, ee_mlp, uid2d, eid2d):
    mesh = plsc.VectorSubcoreMesh(core_axis_name="c", subcore_axis_name="s")
    row_t = jax.ShapeDtypeStruct((B, NF), jnp.float32)

    @functools.partial(
        pl.kernel,
        out_type=(row_t, row_t, row_t, row_t),
        mesh=mesh,
        compiler_params=pltpu.CompilerParams(use_tc_tiling_on_sc=False),
        scratch_types=[
            pltpu.VMEM((_NCHUNK, _CHUNK), jnp.int32),
            pltpu.VMEM((_NCHUNK, _CHUNK), jnp.int32),
            pltpu.VMEM((_BPW, NF), jnp.float32),
            pltpu.VMEM((_BPW, NF), jnp.float32),
            pltpu.VMEM((_BPW, NF), jnp.float32),
            pltpu.VMEM((_BPW, NF), jnp.float32),
            pltpu.SemaphoreType.DMA,
        ],
    )
    def k(ug_hbm, eg_hbm, um_hbm, em_hbm, uid_hbm, eid_hbm,
          oug_hbm, oeg_hbm, oum_hbm, oem_hbm,
          uidx_v, eidx_v, ug_v, eg_v, um_v, em_v, sem):
        wid = lax.axis_index("s") * _NC + lax.axis_index("c")
        base = wid * _BPW
        pltpu.sync_copy(uid_hbm.at[pl.ds(wid * _NCHUNK, _NCHUNK)], uidx_v)
        pltpu.sync_copy(eid_hbm.at[pl.ds(wid * _NCHUNK, _NCHUNK)], eidx_v)
        copies = []
        for j in range(_NCHUNK):
            dst = pl.ds(j * _CHUNK, _CHUNK)
            copies.append(pltpu.async_copy(
                ug_hbm.at[uidx_v.at[j]], ug_v.at[dst], sem))
            copies.append(pltpu.async_copy(
                eg_hbm.at[eidx_v.at[j]], eg_v.at[dst], sem))
            copies.append(pltpu.async_copy(
                um_hbm.at[uidx_v.at[j]], um_v.at[dst], sem))
            copies.append(pltpu.async_copy(
                em_hbm.at[eidx_v.at[j]], em_v.at[dst], sem))
        for c in copies:
            c.wait()
        out_slice = pl.ds(base, _BPW)
        pltpu.sync_copy(ug_v, oug_hbm.at[out_slice])
        pltpu.sync_copy(eg_v, oeg_hbm.at[out_slice])
        pltpu.sync_copy(um_v, oum_hbm.at[out_slice])
        pltpu.sync_copy(em_v, oem_hbm.at[out_slice])

    return k(ue_gmf, ee_gmf, ue_mlp, ee_mlp, uid2d, eid2d)


def _tc_dense(ug, eg, um, em, w1t, b1, gs1, bt1, w2t, b2, gs2, bt2,
              w3t, b3, gs3, bt3, wp, bp):
    blk = 2048
    grid = B // blk

    def body(ug_ref, eg_ref, um_ref, em_ref, w1_ref, b1_ref, gs1_ref, bt1_ref,
             w2_ref, b2_ref, gs2_ref, bt2_ref, w3_ref, b3_ref, gs3_ref,
             bt3_ref, wp_ref, bp_ref, out_ref):
        x = jnp.concatenate([um_ref[...], em_ref[...]], axis=1)
        h = jnp.dot(x, w1_ref[...], preferred_element_type=jnp.float32)
        h = jnp.maximum(h + b1_ref[...], 0.0) * gs1_ref[...] + bt1_ref[...]
        h = jnp.dot(h, w2_ref[...], preferred_element_type=jnp.float32)
        h = jnp.maximum(h + b2_ref[...], 0.0) * gs2_ref[...] + bt2_ref[...]
        h = jnp.dot(h, w3_ref[...], preferred_element_type=jnp.float32)
        h = jnp.maximum(h + b3_ref[...], 0.0) * gs3_ref[...] + bt3_ref[...]
        gmf = ug_ref[...] * eg_ref[...]
        wp_row = wp_ref[...]
        logit = (jnp.sum(gmf * wp_row[:, :NF], axis=1, keepdims=True)
                 + jnp.sum(h * wp_row[:, NF:], axis=1, keepdims=True)
                 + bp_ref[...])
        out_ref[...] = jax.nn.sigmoid(logit)

    def row_spec(shape):
        return pl.BlockSpec((blk,) + shape[1:], lambda i: (i,) + (0,) * (len(shape) - 1))

    def rep_spec(shape):
        return pl.BlockSpec(shape, lambda i: (0,) * len(shape))

    ins = [ug, eg, um, em, w1t, b1, gs1, bt1, w2t, b2, gs2, bt2,
           w3t, b3, gs3, bt3, wp, bp]
    in_specs = [row_spec(a.shape) if a.shape[0] == B else rep_spec(a.shape)
                for a in ins]
    return pl.pallas_call(
        body,
        grid=(grid,),
        in_specs=in_specs,
        out_specs=pl.BlockSpec((blk, 1), lambda i: (i, 0)),
        out_shape=jax.ShapeDtypeStruct((B, 1), jnp.float32),
    )(*ins)


def kernel(user_ids, exercise_ids, ue_gmf, ee_gmf, ue_mlp, ee_mlp,
           W1, b1, g1, bt1, W2, b2, g2, bt2, W3, b3, g3, bt3, Wp, bp):
    uid2d = user_ids.astype(jnp.int32).reshape(_NW * _NCHUNK, _CHUNK)
    eid2d = exercise_ids.astype(jnp.int32).reshape(_NW * _NCHUNK, _CHUNK)
    ug, eg, um, em = _sc_gather4(ue_gmf, ee_gmf, ue_mlp, ee_mlp, uid2d, eid2d)

    s = 1.0 / jnp.sqrt(jnp.float32(1.0 + EPS))
    args = (ug, eg, um, em,
            W1.T, b1.reshape(1, -1), (g1 * s).reshape(1, -1), bt1.reshape(1, -1),
            W2.T, b2.reshape(1, -1), (g2 * s).reshape(1, -1), bt2.reshape(1, -1),
            W3.T, b3.reshape(1, -1), (g3 * s).reshape(1, -1), bt3.reshape(1, -1),
            Wp, bp.reshape(1, 1))
    out = _tc_dense(*args)
    return out.reshape(B)
